# baseline (device time: 41539 ns/iter reference)
import jax
import jax.numpy as jnp
from jax import lax
from jax.experimental import pallas as pl
from jax.experimental.pallas import tpu as pltpu

N_DEV = 4
B, SQ, SKV, DH = 2, 512, 512, 64
H_PER = 8
D_MODEL = 768
D_SHARD = H_PER * DH
WINDOW = 128
ROWS = B * SQ
CHUNK = ROWS // N_DEV
BAND = 384
LOG2E = 1.4426950408889634


def kernel(x, Wq, K_ext, V_ext, Wo):
    x = x.reshape(ROWS, D_MODEL).astype(jnp.bfloat16)
    Wq = Wq.astype(jnp.bfloat16)
    Wo = Wo.astype(jnp.bfloat16)
    K = K_ext.astype(jnp.bfloat16).transpose(0, 2, 1, 3)
    K = K.reshape(B * H_PER, SKV, DH)
    V = V_ext.astype(jnp.bfloat16).transpose(0, 2, 1, 3)
    V = V.reshape(B * H_PER, SKV, DH)

    qs_v = jnp.array([0, CHUNK], jnp.int32)[:, None, None]
    row_i = lax.broadcasted_iota(jnp.int32, (2, CHUNK, BAND), 1)
    col_i = lax.broadcasted_iota(jnp.int32, (2, CHUNK, BAND), 2)
    bias = jnp.where(
        jnp.abs((qs_v + row_i) - (qs_v // 2 + col_i)) <= WINDOW, 0.0, -1e9
    ).astype(jnp.float32)

    def body(x_ref, wq_ref, k_ref, v_ref, wo_ref, bias_ref, out_ref,
             ctx_s, rs_send, rs_recv,
             rs_send_sems, rs_recv_sems, ag_send_sems, ag_recv_sems):
        my = lax.axis_index("i")

        barrier_sem = pltpu.get_barrier_semaphore()
        for s in range(1, N_DEV):
            peer = lax.rem(my + s, N_DEV)
            pl.semaphore_signal(barrier_sem, inc=1, device_id=(peer,),
                                device_id_type=pl.DeviceIdType.MESH)
        pl.semaphore_wait(barrier_sem, N_DEV - 1)

        rs_rdmas = []

        def compute_rows(c, r0, nrows, bias_rows):
            qs = lax.rem(c, 2) * CHUNK
            ks = lax.div(qs, 2)
            fb0 = lax.div(c, 2) * H_PER
            q_chunk = (lax.dot_general(
                x_ref[pl.ds(c * CHUNK + r0, nrows), :],
                wq_ref[:, pl.ds(my * D_SHARD, D_SHARD)],
                (((1,), (0,)), ((), ())),
                preferred_element_type=jnp.float32,
            ) * (0.125 * LOG2E)).astype(jnp.bfloat16)
            for h in range(H_PER):
                qh = q_chunk[:, h * DH:(h + 1) * DH]
                kh = k_ref[fb0 + h, pl.ds(ks, BAND), :]
                vh = v_ref[fb0 + h, pl.ds(ks, BAND), :]
                scores = lax.dot_general(
                    qh, kh, (((1,), (1,)), ((), ())),
                    preferred_element_type=jnp.float32,
                ) + bias_rows
                w = jnp.exp2(scores)
                wsum = jnp.sum(w, axis=1, keepdims=True)
                ctx_h = lax.dot_general(
                    w.astype(jnp.bfloat16), vh, (((1,), (0,)), ((), ())),
                    preferred_element_type=jnp.float32,
                ) / wsum
                ctx_s[pl.ds(0, nrows), h * DH:(h + 1) * DH] = ctx_h.astype(
                    jnp.bfloat16)
            return lax.dot_general(
                ctx_s[pl.ds(0, nrows), :],
                wo_ref[pl.ds(my * D_SHARD, D_SHARD), :],
                (((1,), (0,)), ((), ())),
                preferred_element_type=jnp.float32,
            )

        HALF = CHUNK // 2
        for t in range(N_DEV - 1):
            c = lax.rem(my + t + 1, N_DEV)
            c_phase = lax.rem(c, 2)
            for half in range(2):
                r0 = half * HALF
                contrib = compute_rows(
                    c, r0, HALF, bias_ref[c_phase, pl.ds(r0, HALF), :])
                rs_send[t, pl.ds(r0, HALF), :] = contrib.astype(jnp.bfloat16)
                rdma = pltpu.make_async_remote_copy(
                    src_ref=rs_send.at[t, pl.ds(r0, HALF), :],
                    dst_ref=rs_recv.at[2 - t, pl.ds(r0, HALF), :],
                    send_sem=rs_send_sems.at[t, half],
                    recv_sem=rs_recv_sems.at[2 - t, half],
                    device_id=(c,),
                    device_id_type=pl.DeviceIdType.MESH,
                )
                rdma.start()
                rs_rdmas.append(rdma)

        for rdma in rs_rdmas:
            rdma.wait_recv()

        my_phase = lax.rem(my, 2)
        ag_rdmas = []
        for half in range(2):
            r0 = half * HALF
            contrib = compute_rows(
                my, r0, HALF, bias_ref[my_phase, pl.ds(r0, HALF), :])
            acc = contrib
            for s in range(N_DEV - 1):
                acc = acc + rs_recv[s, pl.ds(r0, HALF), :].astype(jnp.float32)
            out_half = out_ref.at[pl.ds(my * CHUNK + r0, HALF), :]
            out_ref[pl.ds(my * CHUNK + r0, HALF), :] = acc.astype(jnp.bfloat16)
            for s in range(N_DEV - 1):
                p = lax.rem(my + s + 1, N_DEV)
                rdma = pltpu.make_async_remote_copy(
                    src_ref=out_half,
                    dst_ref=out_half,
                    send_sem=ag_send_sems.at[half, s],
                    recv_sem=ag_recv_sems.at[half, 2 - s],
                    device_id=(p,),
                    device_id_type=pl.DeviceIdType.MESH,
                )
                rdma.start()
                ag_rdmas.append(rdma)

        for rdma in ag_rdmas:
            rdma.wait_recv()

        for rdma in rs_rdmas + ag_rdmas:
            rdma.wait_send()

    out = pl.pallas_call(
        body,
        out_shape=jax.ShapeDtypeStruct((ROWS, D_MODEL), jnp.bfloat16),
        in_specs=[pl.BlockSpec(memory_space=pltpu.VMEM)] * 6,
        out_specs=pl.BlockSpec(memory_space=pltpu.VMEM),
        scratch_shapes=[
            pltpu.VMEM((CHUNK, D_SHARD), jnp.bfloat16),
            pltpu.VMEM((3, CHUNK, D_MODEL), jnp.bfloat16),
            pltpu.VMEM((3, CHUNK, D_MODEL), jnp.bfloat16),
            pltpu.SemaphoreType.DMA((3, 2)),
            pltpu.SemaphoreType.DMA((3, 2)),
            pltpu.SemaphoreType.DMA((2, 3)),
            pltpu.SemaphoreType.DMA((2, 3)),
        ],
        compiler_params=pltpu.CompilerParams(collective_id=0),
    )(x, Wq, K, V, Wo, bias)
    return out.reshape(B, SQ, D_MODEL)


# device time: 38343 ns/iter; 1.0834x vs baseline; 1.0834x over previous
import jax
import jax.numpy as jnp
from jax import lax
from jax.experimental import pallas as pl
from jax.experimental.pallas import tpu as pltpu

N_DEV = 4
B, SQ, SKV, DH = 2, 512, 512, 64
H_PER = 8
D_MODEL = 768
D_SHARD = H_PER * DH
WINDOW = 128
ROWS = B * SQ
CHUNK = ROWS // N_DEV
BAND = 384
LOG2E = 1.4426950408889634


def kernel(x, Wq, K_ext, V_ext, Wo):
    x = x.reshape(ROWS, D_MODEL).astype(jnp.bfloat16)
    Wq = Wq.astype(jnp.bfloat16)
    Wo = Wo.astype(jnp.bfloat16)
    K = K_ext.astype(jnp.bfloat16).transpose(0, 2, 1, 3)
    K = K.reshape(B * H_PER, SKV, DH)
    V = V_ext.astype(jnp.bfloat16).transpose(0, 2, 1, 3)
    V = V.reshape(B * H_PER, SKV, DH)

    qs_v = jnp.array([0, CHUNK], jnp.int32)[:, None, None]
    row_i = lax.broadcasted_iota(jnp.int32, (2, CHUNK, BAND), 1)
    col_i = lax.broadcasted_iota(jnp.int32, (2, CHUNK, BAND), 2)
    bias = jnp.where(
        jnp.abs((qs_v + row_i) - (qs_v // 2 + col_i)) <= WINDOW, 0.0, -1e9
    ).astype(jnp.float32)

    def body(x_ref, wq_ref, k_ref, v_ref, wo_ref, bias_ref, out_ref,
             ctx_s, rs_send, rs_recv,
             rs_send_sems, rs_recv_sems, ag_send_sems, ag_recv_sems):
        my = lax.axis_index("i")

        barrier_sem = pltpu.get_barrier_semaphore()
        for s in range(1, N_DEV):
            peer = lax.rem(my + s, N_DEV)
            pl.semaphore_signal(barrier_sem, inc=1, device_id=(peer,),
                                device_id_type=pl.DeviceIdType.MESH)
        pl.semaphore_wait(barrier_sem, N_DEV - 1)

        rs_rdmas = []

        def compute_rows(c, r0, nrows, bias_rows):
            qs = lax.rem(c, 2) * CHUNK
            ks = lax.div(qs, 2)
            fb0 = lax.div(c, 2) * H_PER
            q_chunk = (lax.dot_general(
                x_ref[pl.ds(c * CHUNK + r0, nrows), :],
                wq_ref[:, pl.ds(my * D_SHARD, D_SHARD)],
                (((1,), (0,)), ((), ())),
                preferred_element_type=jnp.float32,
            ) * (0.125 * LOG2E)).astype(jnp.bfloat16)
            for h in range(H_PER):
                qh = q_chunk[:, h * DH:(h + 1) * DH]
                kh = k_ref[fb0 + h, pl.ds(ks, BAND), :]
                vh = v_ref[fb0 + h, pl.ds(ks, BAND), :]
                scores = lax.dot_general(
                    qh, kh, (((1,), (1,)), ((), ())),
                    preferred_element_type=jnp.float32,
                ) + bias_rows
                w = jnp.exp2(scores)
                wsum = jnp.sum(w, axis=1, keepdims=True)
                ctx_h = lax.dot_general(
                    w.astype(jnp.bfloat16), vh, (((1,), (0,)), ((), ())),
                    preferred_element_type=jnp.float32,
                ) / wsum
                ctx_s[pl.ds(0, nrows), h * DH:(h + 1) * DH] = ctx_h.astype(
                    jnp.bfloat16)
            return lax.dot_general(
                ctx_s[pl.ds(0, nrows), :],
                wo_ref[pl.ds(my * D_SHARD, D_SHARD), :],
                (((1,), (0,)), ((), ())),
                preferred_element_type=jnp.float32,
            )

        for t in range(N_DEV - 1):
            c = lax.rem(my + t + 1, N_DEV)
            contrib = compute_rows(c, 0, CHUNK, bias_ref[lax.rem(c, 2)])
            rs_send[t] = contrib.astype(jnp.bfloat16)
            rdma = pltpu.make_async_remote_copy(
                src_ref=rs_send.at[t],
                dst_ref=rs_recv.at[2 - t],
                send_sem=rs_send_sems.at[t],
                recv_sem=rs_recv_sems.at[2 - t],
                device_id=(c,),
                device_id_type=pl.DeviceIdType.MESH,
            )
            rdma.start()
            rs_rdmas.append(rdma)

        HALF = CHUNK // 2
        my_phase = lax.rem(my, 2)
        ag_rdmas = []
        for half in range(2):
            r0 = half * HALF
            contrib = compute_rows(
                my, r0, HALF, bias_ref[my_phase, pl.ds(r0, HALF), :])
            if half == 0:
                for rdma in rs_rdmas:
                    rdma.wait_recv()
            acc = contrib
            for s in range(N_DEV - 1):
                acc = acc + rs_recv[s, pl.ds(r0, HALF), :].astype(jnp.float32)
            out_half = out_ref.at[pl.ds(my * CHUNK + r0, HALF), :]
            out_ref[pl.ds(my * CHUNK + r0, HALF), :] = acc.astype(jnp.bfloat16)
            for s in range(N_DEV - 1):
                p = lax.rem(my + s + 1, N_DEV)
                rdma = pltpu.make_async_remote_copy(
                    src_ref=out_half,
                    dst_ref=out_half,
                    send_sem=ag_send_sems.at[half, s],
                    recv_sem=ag_recv_sems.at[half, 2 - s],
                    device_id=(p,),
                    device_id_type=pl.DeviceIdType.MESH,
                )
                rdma.start()
                ag_rdmas.append(rdma)

        for rdma in ag_rdmas:
            rdma.wait_recv()

        for rdma in rs_rdmas + ag_rdmas:
            rdma.wait_send()

    out = pl.pallas_call(
        body,
        out_shape=jax.ShapeDtypeStruct((ROWS, D_MODEL), jnp.bfloat16),
        in_specs=[pl.BlockSpec(memory_space=pltpu.VMEM)] * 6,
        out_specs=pl.BlockSpec(memory_space=pltpu.VMEM),
        scratch_shapes=[
            pltpu.VMEM((CHUNK, D_SHARD), jnp.bfloat16),
            pltpu.VMEM((3, CHUNK, D_MODEL), jnp.bfloat16),
            pltpu.VMEM((3, CHUNK, D_MODEL), jnp.bfloat16),
            pltpu.SemaphoreType.DMA((3,)),
            pltpu.SemaphoreType.DMA((3,)),
            pltpu.SemaphoreType.DMA((2, 3)),
            pltpu.SemaphoreType.DMA((2, 3)),
        ],
        compiler_params=pltpu.CompilerParams(collective_id=0),
    )(x, Wq, K, V, Wo, bias)
    return out.reshape(B, SQ, D_MODEL)
